# baseline (speedup 1.0000x reference)
import functools

import jax
import jax.numpy as jnp
import numpy as np
from jax import lax
from jax.experimental import pallas as pl
from jax.experimental.pallas import tpu as pltpu
from jax.experimental.pallas import tpu_sc as plsc

B, L = 16, 65536
LROWS = L // 16
P, J = 30, 17
PP = 32
NT = J * PP
KPW = 1032
CHUNKS = [(0, 128, 4), (128, 128, 8), (256, 128, 12), (384, 128, 16),
          (512, 32, 17)]

_I = np.arange(64)
_K = np.arange(32)
_RECIP_TAB = np.concatenate([
    1.0 / np.maximum(_I, 1),
    1.0 / np.maximum(_K * (_K - 1) / 2.0, 1.0),
]).astype(np.float32)


def _aeloss_body(tags_hbm, kp_hbm, tab_hbm, out_hbm,
                 kp_v, rows_v, lanes_v, w_v, gath_v, tab_v, means_v,
                 out_v, asm_v, out2_v, shared_v, sem):
    c = lax.axis_index("c")
    s = lax.axis_index("s")
    iota = lax.iota(jnp.int32, 16)

    @pl.when(s < 8)
    def _():
        b = c * 8 + s
        pltpu.sync_copy(kp_hbm.at[b], kp_v)
        pltpu.sync_copy(tab_hbm, tab_v)

        maskb = iota < P - 16
        pers_a = iota
        pers_b = jnp.where(maskb, iota + 16, 0)
        zero_i = jnp.zeros((16,), jnp.int32)
        one_i = zero_i + 1
        copies = []
        tab3 = tags_hbm.at[b]
        ci = 0
        for j in range(J):
            jv = zero_i + j
            ia = plsc.load_gather(kp_v, [pers_a, jv, zero_i])
            ib = plsc.load_gather(kp_v, [pers_b, jv, zero_i])
            fa = plsc.load_gather(kp_v, [pers_a, jv, one_i])
            fb = plsc.load_gather(kp_v, [pers_b, jv, one_i])
            rows_v[pl.ds(j * PP, 16)] = jnp.right_shift(ia, 4)
            rows_v[pl.ds(j * PP + 16, 16)] = jnp.right_shift(ib, 4)
            lanes_v[pl.ds(j * PP, 16)] = jnp.bitwise_and(ia, 15)
            lanes_v[pl.ds(j * PP + 16, 16)] = jnp.bitwise_and(ib, 15)
            w_v[pl.ds(j * PP, 16)] = fa.astype(jnp.float32)
            w_v[pl.ds(j * PP + 16, 16)] = jnp.where(
                maskb, fb, 0).astype(jnp.float32)
            while ci < len(CHUNKS) and CHUNKS[ci][2] == j + 1:
                start, nsl, _ = CHUNKS[ci]
                copies.append(pltpu.async_copy(
                    tab3.at[rows_v.at[pl.ds(start, nsl)]],
                    gath_v.at[pl.ds(start, nsl)], sem))
                ci += 1

        na = jnp.zeros((16,), jnp.float32)
        nb = jnp.zeros((16,), jnp.float32)
        sa = jnp.zeros((16,), jnp.float32)
        sb = jnp.zeros((16,), jnp.float32)
        qa = jnp.zeros((16,), jnp.float32)
        qb = jnp.zeros((16,), jnp.float32)
        ci = 0
        for j in range(J):
            while ci < len(CHUNKS) and CHUNKS[ci][0] <= j * PP:
                copies[ci].wait()
                ci += 1
            ta = plsc.load_gather(
                gath_v, [j * PP + iota, lanes_v[pl.ds(j * PP, 16)]])
            tb = plsc.load_gather(
                gath_v, [j * PP + 16 + iota, lanes_v[pl.ds(j * PP + 16, 16)]])
            wa = w_v[pl.ds(j * PP, 16)]
            wb = w_v[pl.ds(j * PP + 16, 16)]
            twa = ta * wa
            twb = tb * wb
            na = na + wa
            nb = nb + wb
            sa = sa + twa
            sb = sb + twb
            qa = qa + ta * twa
            qb = qb + tb * twb
        for cp in copies[ci:]:
            cp.wait()

        nia = na.astype(jnp.int32)
        nib = nb.astype(jnp.int32)
        rna = plsc.load_gather(tab_v, [nia])
        rnb = plsc.load_gather(tab_v, [nib])
        ma = sa * rna
        mb = sb * rnb
        va = qa - 2.0 * ma * sa + ma * ma * na
        vb = qb - 2.0 * mb * sb + mb * mb * nb

        has_a = nia > 0
        has_b = nib > 0
        zero = jnp.zeros((16,), jnp.float32)
        ma = jnp.where(has_a, ma, zero)
        mb = jnp.where(has_b, mb, zero)
        pushv = jnp.where(has_a, va * rna, zero) + jnp.where(has_b, vb * rnb, zero)
        push_acc = jnp.sum(pushv)
        cur = (plsc.all_reduce_population_count(has_a)
               + plsc.all_reduce_population_count(has_b))

        sent_a = (iota + 1).astype(jnp.float32) * 1e4
        sent_b = (iota + 17).astype(jnp.float32) * 1e4
        means_v[pl.ds(0, 16)] = jnp.where(iota < cur, ma, sent_a)
        means_v[pl.ds(16, 16)] = jnp.where(iota + 16 < cur, mb, sent_b)
        ma = means_v[pl.ds(0, 16)]
        mb = means_v[pl.ds(16, 16)]

        kf = cur.astype(jnp.float32)
        acc = jnp.zeros((16,), jnp.float32)
        for j in range(P):
            bj = plsc.load_gather(means_v, [jnp.full((16,), j, jnp.int32)])
            da = ma - bj
            db = mb - bj
            acc = acc + jnp.exp(-(da * da)) + jnp.exp(-(db * db))
        total = jnp.sum(acc)

        rk = plsc.load_gather(tab_v, [cur])
        rp = plsc.load_gather(tab_v, [cur + 64])
        pull = (total - jnp.float32(P)) * 0.5
        pull = jnp.where(cur > 1, pull * rp, zero + pull)
        pull = pull * 0.5
        push = jnp.where(cur > 0, push_acc * rk, zero + push_acc)

        out_v[...] = jnp.where(iota == 0, pull,
                               jnp.where(iota == 1, push, 0.0))
        pltpu.sync_copy(out_v, shared_v.at[s])

    plsc.subcore_barrier()

    @pl.when(s == 0)
    def _():
        pltpu.sync_copy(shared_v, asm_v)
        r = plsc.load_gather(
            asm_v, [jnp.right_shift(iota, 1), jnp.bitwise_and(iota, 1)])
        out2_v[...] = r
        pltpu.sync_copy(out2_v, out_hbm.at[pl.ds(c * 16, 16)])


_aeloss = functools.partial(
    pl.kernel,
    out_type=jax.ShapeDtypeStruct((2 * B,), jnp.float32),
    mesh=plsc.VectorSubcoreMesh(core_axis_name="c", subcore_axis_name="s"),
    compiler_params=pltpu.CompilerParams(
        needs_layout_passes=False, use_tc_tiling_on_sc=False),
    scratch_types=[
        pltpu.VMEM((P, J, 2), jnp.int32),
        pltpu.VMEM((NT,), jnp.int32),
        pltpu.VMEM((NT,), jnp.int32),
        pltpu.VMEM((NT,), jnp.float32),
        pltpu.VMEM((NT, 16), jnp.float32),
        pltpu.VMEM((96,), jnp.float32),
        pltpu.VMEM((32,), jnp.float32),
        pltpu.VMEM((16,), jnp.float32),
        pltpu.VMEM((8, 16), jnp.float32),
        pltpu.VMEM((16,), jnp.float32),
        pltpu.VMEM_SHARED((8, 16), jnp.float32),
        pltpu.SemaphoreType.DMA,
    ],
)(_aeloss_body)


@jax.jit
def kernel(tags, keypoints):
    tags3 = tags.reshape(B, LROWS, 16)
    out = _aeloss(tags3, keypoints, jnp.asarray(_RECIP_TAB))
    return out.reshape(B, 2)


# speedup vs baseline: 1.4798x; 1.4798x over previous
import functools

import jax
import jax.numpy as jnp
import numpy as np
from jax import lax
from jax.experimental import pallas as pl
from jax.experimental.pallas import tpu as pltpu
from jax.experimental.pallas import tpu_sc as plsc

B, L = 16, 65536
LROWS = L // 16
P, J = 30, 17
PP = 32
NT = J * PP
KPW = 1088
CHUNKS = [(0, 128, 4), (128, 128, 8), (256, 128, 12), (384, 128, 16),
          (512, 32, 17)]

_I = np.arange(64)
_K = np.arange(32)
_RECIP_TAB = np.concatenate([
    1.0 / np.maximum(_I, 1),
    1.0 / np.maximum(_K * (_K - 1) / 2.0, 1.0),
]).astype(np.float32)


def _aeloss_body(tags_hbm, kp_hbm, tab_hbm, out_hbm,
                 kp_v, rows_v, lanes_v, w_v, gath_v, tab_v, means_v,
                 out_v, asm_v, out2_v, shared_v, sem):
    c = lax.axis_index("c")
    s = lax.axis_index("s")
    iota = lax.iota(jnp.int32, 16)

    @pl.when(s < 8)
    def _():
        b = c * 8 + s
        pltpu.sync_copy(kp_hbm.at[b], kp_v)
        pltpu.sync_copy(tab_hbm, tab_v)

        base_a = 34 * iota
        base_b = 34 * (iota + 16)
        copies = []
        tab3 = tags_hbm.at[b]
        ci = 0
        for j in range(J):
            ia = plsc.load_gather(kp_v, [base_a + 2 * j])
            ib = plsc.load_gather(kp_v, [base_b + 2 * j])
            fa = plsc.load_gather(kp_v, [base_a + 2 * j + 1])
            fb = plsc.load_gather(kp_v, [base_b + 2 * j + 1])
            rows_v[pl.ds(j * PP, 16)] = jnp.right_shift(ia, 4)
            rows_v[pl.ds(j * PP + 16, 16)] = jnp.right_shift(ib, 4)
            lanes_v[pl.ds(j * PP, 16)] = jnp.bitwise_and(ia, 15)
            lanes_v[pl.ds(j * PP + 16, 16)] = jnp.bitwise_and(ib, 15)
            w_v[pl.ds(j * PP, 16)] = fa.astype(jnp.float32)
            w_v[pl.ds(j * PP + 16, 16)] = fb.astype(jnp.float32)
            while ci < len(CHUNKS) and CHUNKS[ci][2] == j + 1:
                start, nsl, _ = CHUNKS[ci]
                copies.append(pltpu.async_copy(
                    tab3.at[rows_v.at[pl.ds(start, nsl)]],
                    gath_v.at[pl.ds(start, nsl)], sem))
                ci += 1

        def joint_step(j, carry):
            na, nb, sa, sb, qa, qb = carry
            o = pl.multiple_of(j * PP, 8)
            ta = plsc.load_gather(
                gath_v, [j * PP + iota, lanes_v[pl.ds(o, 16)]])
            tb = plsc.load_gather(
                gath_v, [j * PP + 16 + iota,
                         lanes_v[pl.ds(o + 16, 16)]])
            wa = w_v[pl.ds(o, 16)]
            wb = w_v[pl.ds(o + 16, 16)]
            twa = ta * wa
            twb = tb * wb
            return (na + wa, nb + wb, sa + twa, sb + twb,
                    qa + ta * twa, qb + tb * twb)

        z = jnp.zeros((16,), jnp.float32)
        for cp in copies[:4]:
            cp.wait()
        na, nb, sa, sb, qa, qb = lax.fori_loop(
            0, 16, joint_step, (z, z, z, z, z, z), unroll=4)
        copies[4].wait()
        na, nb, sa, sb, qa, qb = joint_step(16, (na, nb, sa, sb, qa, qb))

        nia = na.astype(jnp.int32)
        nib = nb.astype(jnp.int32)
        rna = plsc.load_gather(tab_v, [nia])
        rnb = plsc.load_gather(tab_v, [nib])
        ma = sa * rna
        mb = sb * rnb
        va = qa - 2.0 * ma * sa + ma * ma * na
        vb = qb - 2.0 * mb * sb + mb * mb * nb

        has_a = nia > 0
        has_b = nib > 0
        zero = jnp.zeros((16,), jnp.float32)
        ma = jnp.where(has_a, ma, zero)
        mb = jnp.where(has_b, mb, zero)
        pushv = jnp.where(has_a, va * rna, zero) + jnp.where(has_b, vb * rnb, zero)
        push_acc = jnp.sum(pushv)
        cur = (plsc.all_reduce_population_count(has_a)
               + plsc.all_reduce_population_count(has_b))

        sent_a = (iota + 1).astype(jnp.float32) * 1e4
        sent_b = (iota + 17).astype(jnp.float32) * 1e4
        means_v[pl.ds(0, 16)] = jnp.where(iota < cur, ma, sent_a)
        means_v[pl.ds(16, 16)] = jnp.where(iota + 16 < cur, mb, sent_b)
        ma = means_v[pl.ds(0, 16)]
        mb = means_v[pl.ds(16, 16)]

        def pair_step(j, acc):
            bj = plsc.load_gather(means_v, [jnp.zeros((16,), jnp.int32) + j])
            da = ma - bj
            db = mb - bj
            return acc + jnp.exp(-(da * da)) + jnp.exp(-(db * db))

        acc = lax.fori_loop(0, P, pair_step, zero, unroll=5)
        total = jnp.sum(acc)

        rk = plsc.load_gather(tab_v, [cur])
        rp = plsc.load_gather(tab_v, [cur + 64])
        pull = (total - jnp.float32(P)) * 0.5
        pull = jnp.where(cur > 1, pull * rp, zero + pull)
        pull = pull * 0.5
        push = jnp.where(cur > 0, push_acc * rk, zero + push_acc)

        out_v[...] = jnp.where(iota == 0, pull,
                               jnp.where(iota == 1, push, 0.0))
        pltpu.sync_copy(out_v, shared_v.at[s])

    plsc.subcore_barrier()

    @pl.when(s == 0)
    def _():
        pltpu.sync_copy(shared_v, asm_v)
        r = plsc.load_gather(
            asm_v, [jnp.right_shift(iota, 1), jnp.bitwise_and(iota, 1)])
        out2_v[...] = r
        pltpu.sync_copy(out2_v, out_hbm.at[pl.ds(c * 16, 16)])


_aeloss = functools.partial(
    pl.kernel,
    out_type=jax.ShapeDtypeStruct((2 * B,), jnp.float32),
    mesh=plsc.VectorSubcoreMesh(core_axis_name="c", subcore_axis_name="s"),
    compiler_params=pltpu.CompilerParams(
        needs_layout_passes=False, use_tc_tiling_on_sc=False),
    scratch_types=[
        pltpu.VMEM((KPW,), jnp.int32),
        pltpu.VMEM((NT,), jnp.int32),
        pltpu.VMEM((NT,), jnp.int32),
        pltpu.VMEM((NT,), jnp.float32),
        pltpu.VMEM((NT, 16), jnp.float32),
        pltpu.VMEM((96,), jnp.float32),
        pltpu.VMEM((32,), jnp.float32),
        pltpu.VMEM((16,), jnp.float32),
        pltpu.VMEM((8, 16), jnp.float32),
        pltpu.VMEM((16,), jnp.float32),
        pltpu.VMEM_SHARED((8, 16), jnp.float32),
        pltpu.SemaphoreType.DMA,
    ],
)(_aeloss_body)


@jax.jit
def kernel(tags, keypoints):
    tags3 = tags.reshape(B, LROWS, 16)
    kp = jnp.pad(keypoints.reshape(B, P * J * 2), ((0, 0), (0, KPW - P * J * 2)))
    out = _aeloss(tags3, kp, jnp.asarray(_RECIP_TAB))
    return out.reshape(B, 2)


# speedup vs baseline: 1.5664x; 1.0585x over previous
import functools

import jax
import jax.numpy as jnp
import numpy as np
from jax import lax
from jax.experimental import pallas as pl
from jax.experimental.pallas import tpu as pltpu
from jax.experimental.pallas import tpu_sc as plsc

B, L = 16, 65536
LROWS = L // 16
P, J = 30, 17
PP = 32
NT = J * PP
KPW = 1088
CHUNKS = [(0, 128, 4), (128, 128, 8), (256, 128, 12), (384, 128, 16),
          (512, 32, 17)]

def _recip(x):
    y = plsc.bitcast(jnp.int32(0x7EF311C3) - plsc.bitcast(x, jnp.int32),
                     jnp.float32)
    for _ in range(3):
        y = y * (2.0 - x * y)
    return y


def _aeloss_body(tags_hbm, kp_hbm, out_hbm,
                 kp_v, rows_v, lanes_v, w_v, gath_v, means_v,
                 out_v, asm_v, out2_v, shared_v, sem):
    c = lax.axis_index("c")
    s = lax.axis_index("s")
    iota = lax.iota(jnp.int32, 16)

    @pl.when(s < 8)
    def _():
        b = c * 8 + s
        pltpu.sync_copy(kp_hbm.at[b], kp_v)

        base_a = 34 * iota
        base_b = 34 * (iota + 16)
        copies = []
        tab3 = tags_hbm.at[b]
        ci = 0
        for j in range(J):
            ia = plsc.load_gather(kp_v, [base_a + 2 * j])
            ib = plsc.load_gather(kp_v, [base_b + 2 * j])
            fa = plsc.load_gather(kp_v, [base_a + 2 * j + 1])
            fb = plsc.load_gather(kp_v, [base_b + 2 * j + 1])
            rows_v[pl.ds(j * PP, 16)] = jnp.right_shift(ia, 4)
            rows_v[pl.ds(j * PP + 16, 16)] = jnp.right_shift(ib, 4)
            lanes_v[pl.ds(j * PP, 16)] = jnp.bitwise_and(ia, 15)
            lanes_v[pl.ds(j * PP + 16, 16)] = jnp.bitwise_and(ib, 15)
            w_v[pl.ds(j * PP, 16)] = fa.astype(jnp.float32)
            w_v[pl.ds(j * PP + 16, 16)] = fb.astype(jnp.float32)
            while ci < len(CHUNKS) and CHUNKS[ci][2] == j + 1:
                start, nsl, _ = CHUNKS[ci]
                copies.append(pltpu.async_copy(
                    tab3.at[rows_v.at[pl.ds(start, nsl)]],
                    gath_v.at[pl.ds(start, nsl)], sem))
                ci += 1

        def joint_step(j, carry):
            na, nb, sa, sb, qa, qb = carry
            o = pl.multiple_of(j * PP, 8)
            ta = plsc.load_gather(
                gath_v, [j * PP + iota, lanes_v[pl.ds(o, 16)]])
            tb = plsc.load_gather(
                gath_v, [j * PP + 16 + iota,
                         lanes_v[pl.ds(o + 16, 16)]])
            wa = w_v[pl.ds(o, 16)]
            wb = w_v[pl.ds(o + 16, 16)]
            twa = ta * wa
            twb = tb * wb
            return (na + wa, nb + wb, sa + twa, sb + twb,
                    qa + ta * twa, qb + tb * twb)

        z = jnp.zeros((16,), jnp.float32)
        for cp in copies[:4]:
            cp.wait()
        na, nb, sa, sb, qa, qb = lax.fori_loop(
            0, 16, joint_step, (z, z, z, z, z, z), unroll=4)
        copies[4].wait()
        na, nb, sa, sb, qa, qb = joint_step(16, (na, nb, sa, sb, qa, qb))

        nia = na.astype(jnp.int32)
        nib = nb.astype(jnp.int32)
        rna = _recip(jnp.maximum(na, 1.0))
        rnb = _recip(jnp.maximum(nb, 1.0))
        ma = sa * rna
        mb = sb * rnb
        va = qa - 2.0 * ma * sa + ma * ma * na
        vb = qb - 2.0 * mb * sb + mb * mb * nb

        has_a = nia > 0
        has_b = nib > 0
        zero = jnp.zeros((16,), jnp.float32)
        ma = jnp.where(has_a, ma, zero)
        mb = jnp.where(has_b, mb, zero)
        pushv = jnp.where(has_a, va * rna, zero) + jnp.where(has_b, vb * rnb, zero)
        push_acc = jnp.sum(pushv)
        cur = (plsc.all_reduce_population_count(has_a)
               + plsc.all_reduce_population_count(has_b))

        sent_a = (iota + 1).astype(jnp.float32) * 1e4
        sent_b = (iota + 17).astype(jnp.float32) * 1e4
        means_v[pl.ds(0, 16)] = jnp.where(iota < cur, ma, sent_a)
        means_v[pl.ds(16, 16)] = jnp.where(iota + 16 < cur, mb, sent_b)
        ma = means_v[pl.ds(0, 16)]
        mb = means_v[pl.ds(16, 16)]

        def pair_step(j, acc):
            bj = plsc.load_gather(means_v, [jnp.zeros((16,), jnp.int32) + j])
            da = ma - bj
            db = mb - bj
            return acc + jnp.exp(-(da * da)) + jnp.exp(-(db * db))

        acc = lax.fori_loop(0, P, pair_step, zero, unroll=5)
        total = jnp.sum(acc)

        kf = cur.astype(jnp.float32)
        kf1 = jnp.maximum(kf, 1.0)
        rk = _recip(kf1)
        rp = _recip(jnp.maximum(kf1 * (kf1 - 1.0) * 0.5, 1.0))
        pull = (total - jnp.float32(P)) * 0.5
        pull = jnp.where(cur > 1, pull * rp, zero + pull)
        pull = pull * 0.5
        push = jnp.where(cur > 0, push_acc * rk, zero + push_acc)

        out_v[...] = jnp.where(iota == 0, pull,
                               jnp.where(iota == 1, push, 0.0))
        pltpu.sync_copy(out_v, shared_v.at[s])

    plsc.subcore_barrier()

    @pl.when(s == 0)
    def _():
        pltpu.sync_copy(shared_v, asm_v)
        r = plsc.load_gather(
            asm_v, [jnp.bitwise_and(iota, 7), jnp.right_shift(iota, 3)])
        out2_v[...] = r
        pltpu.sync_copy(out2_v.at[pl.ds(0, 8)], out_hbm.at[pl.ds(c * 8, 8)])
        pltpu.sync_copy(out2_v.at[pl.ds(8, 8)],
                        out_hbm.at[pl.ds(B + c * 8, 8)])


_aeloss = functools.partial(
    pl.kernel,
    out_type=jax.ShapeDtypeStruct((2 * B,), jnp.float32),
    mesh=plsc.VectorSubcoreMesh(core_axis_name="c", subcore_axis_name="s"),
    compiler_params=pltpu.CompilerParams(
        needs_layout_passes=False, use_tc_tiling_on_sc=False),
    scratch_types=[
        pltpu.VMEM((KPW,), jnp.int32),
        pltpu.VMEM((NT,), jnp.int32),
        pltpu.VMEM((NT,), jnp.int32),
        pltpu.VMEM((NT,), jnp.float32),
        pltpu.VMEM((NT, 16), jnp.float32),
        pltpu.VMEM((32,), jnp.float32),
        pltpu.VMEM((16,), jnp.float32),
        pltpu.VMEM((8, 16), jnp.float32),
        pltpu.VMEM((16,), jnp.float32),
        pltpu.VMEM_SHARED((8, 16), jnp.float32),
        pltpu.SemaphoreType.DMA,
    ],
)(_aeloss_body)


@jax.jit
def kernel(tags, keypoints):
    tags3 = tags.reshape(B, LROWS, 16)
    kp = jnp.pad(keypoints.reshape(B, P * J * 2), ((0, 0), (0, KPW - P * J * 2)))
    out = _aeloss(tags3, kp)
    return jnp.transpose(out.reshape(2, B), (1, 0))
